# baseline (speedup 1.0000x reference)
import functools

import jax
import jax.numpy as jnp
from jax import lax
from jax.experimental import pallas as pl
from jax.experimental.pallas import tpu as pltpu
from jax.experimental.pallas import tpu_sc as plsc

N_NODES = 10000
N_EDGES = 320000
D = 128

NC = 2
NS = 16
NW = NC * NS
CHUNK = 100
N_CHUNKS = 104
EPW = N_CHUNKS * CHUNK
SUPER = 8
N_SUPER = N_CHUNKS // SUPER
N_PAD = 10240
ZCHUNK = 80
ROWS_PER_TILE = N_PAD // NS
PAD_DST = N_NODES

_mesh = plsc.VectorSubcoreMesh(core_axis_name="c", subcore_axis_name="s")


@functools.partial(
    pl.kernel,
    out_type=jax.ShapeDtypeStruct((NC, N_PAD, D), jnp.float32),
    mesh=_mesh,
    scratch_types=[
        pltpu.VMEM((SUPER, CHUNK), jnp.int32),
        pltpu.VMEM((SUPER, CHUNK), jnp.int32),
        pltpu.VMEM((2, CHUNK, D), jnp.float32),
        pltpu.VMEM_SHARED((N_PAD, D), jnp.float32),
        pltpu.SemaphoreType.DMA,
        pltpu.SemaphoreType.DMA,
    ],
)
def _sc_aggregate(x_hbm, src_hbm, dst_hbm, out_hbm, src_v, dst_v, rows_v, acc,
                  sem0, sem1):
    cid = lax.axis_index("c")
    sid = lax.axis_index("s")
    wid = cid * NS + sid

    zeros16 = jnp.zeros((16,), jnp.float32)

    def zero_body(i, _):
        rows_v[0, i // (D // 16), pl.ds((i % (D // 16)) * 16, 16)] = zeros16
        return 0

    lax.fori_loop(0, ZCHUNK * (D // 16), zero_body, 0)

    r0 = sid * ROWS_PER_TILE
    for t in range(ROWS_PER_TILE // ZCHUNK):
        pltpu.sync_copy(rows_v.at[0, pl.ds(0, ZCHUNK)],
                        acc.at[pl.ds(r0 + t * ZCHUNK, ZCHUNK)])

    plsc.subcore_barrier()

    sems = (sem0, sem1)

    def super_body(s, _):
        pltpu.sync_copy(src_hbm.at[wid, pl.ds(s * SUPER, SUPER)], src_v)
        pltpu.sync_copy(dst_hbm.at[wid, pl.ds(s * SUPER, SUPER)], dst_v)
        pltpu.async_copy(x_hbm.at[src_v.at[0]], rows_v.at[0], sems[0])
        for k in range(SUPER):
            cur, nxt = k % 2, (k + 1) % 2
            pltpu.make_async_copy(
                x_hbm.at[src_v.at[k]], rows_v.at[cur], sems[cur]).wait()
            if k + 1 < SUPER:
                pltpu.async_copy(
                    x_hbm.at[src_v.at[k + 1]], rows_v.at[nxt], sems[nxt])
            pltpu.sync_copy(rows_v.at[cur], acc.at[dst_v.at[k]], add=True)
        return 0

    lax.fori_loop(0, N_SUPER, super_body, 0)

    plsc.subcore_barrier()

    pltpu.sync_copy(acc.at[pl.ds(r0, ROWS_PER_TILE)],
                    out_hbm.at[cid, pl.ds(r0, ROWS_PER_TILE)])


BR = 1000


def _mlp_body(p_ref, x_ref, w1t_ref, b1_ref, w2t_ref, b2_ref, o_ref):
    h = p_ref[0] + p_ref[1] + x_ref[...]
    h1 = jnp.dot(h, w1t_ref[...], preferred_element_type=jnp.float32) + b1_ref[...]
    o_ref[...] = jnp.dot(h1, w2t_ref[...], preferred_element_type=jnp.float32) + b2_ref[...]


_mlp_call = pl.pallas_call(
    _mlp_body,
    out_shape=jax.ShapeDtypeStruct((N_NODES, D), jnp.float32),
    grid=(N_NODES // BR,),
    in_specs=[
        pl.BlockSpec((NC, BR, D), lambda i: (0, i, 0)),
        pl.BlockSpec((BR, D), lambda i: (i, 0)),
        pl.BlockSpec((D, D), lambda i: (0, 0)),
        pl.BlockSpec((1, D), lambda i: (0, 0)),
        pl.BlockSpec((D, D), lambda i: (0, 0)),
        pl.BlockSpec((1, D), lambda i: (0, 0)),
    ],
    out_specs=pl.BlockSpec((BR, D), lambda i: (i, 0)),
)


def kernel(x, edge_index, W1, b1, W2, b2):
    ei = edge_index.astype(jnp.int32)
    per_w = N_EDGES // NW
    dst = ei[0].reshape(NW, per_w)
    src = ei[1].reshape(NW, per_w)
    pad_rows = PAD_DST + jnp.arange(NW, dtype=jnp.int32)[:, None]
    dst = jnp.concatenate(
        [dst, jnp.broadcast_to(pad_rows, (NW, EPW - per_w))], axis=1)
    src = jnp.pad(src, ((0, 0), (0, EPW - per_w)), constant_values=0)
    dst = dst.reshape(NW, N_CHUNKS, CHUNK)
    src = src.reshape(NW, N_CHUNKS, CHUNK)
    partials = _sc_aggregate(x, src, dst)
    return _mlp_call(partials, x, W1.T, b1.reshape(1, D), W2.T, b2.reshape(1, D))


# speedup vs baseline: 2.8931x; 2.8931x over previous
import functools

import jax
import jax.numpy as jnp
from jax import lax
from jax.experimental import pallas as pl
from jax.experimental.pallas import tpu as pltpu
from jax.experimental.pallas import tpu_sc as plsc

N_NODES = 10000
N_EDGES = 320000
D = 128

NC = 2
NS = 16
NW = NC * NS
CHUNK = 100
N_CHUNKS = 100
EPW = N_CHUNKS * CHUNK
SUPER = 8
N_SUPER = N_CHUNKS // SUPER
N_PAD = 10240
ZCHUNK = 80
ROWS_PER_TILE = N_PAD // NS
PAD_DST = N_NODES

_mesh = plsc.VectorSubcoreMesh(core_axis_name="c", subcore_axis_name="s")


@functools.partial(
    pl.kernel,
    out_type=jax.ShapeDtypeStruct((NC, N_PAD, D), jnp.float32),
    mesh=_mesh,
    scratch_types=[
        pltpu.VMEM((N_CHUNKS, CHUNK), jnp.int32),
        pltpu.VMEM((N_CHUNKS, CHUNK), jnp.int32),
        pltpu.VMEM((CHUNK, D), jnp.float32),
        pltpu.VMEM_SHARED((N_PAD, D), jnp.float32),
        pltpu.SemaphoreType.DMA,
        pltpu.SemaphoreType.DMA,
    ],
)
def _sc_aggregate(x_hbm, src_hbm, dst_hbm, out_hbm, src_v, dst_v, rows_v, acc,
                  sem0, sem1):
    cid = lax.axis_index("c")
    sid = lax.axis_index("s")
    wid = cid * NS + sid

    zeros16 = jnp.zeros((16,), jnp.float32)

    def zero_body(i, _):
        rows_v[i // (D // 16), pl.ds((i % (D // 16)) * 16, 16)] = zeros16
        return 0

    lax.fori_loop(0, ZCHUNK * (D // 16), zero_body, 0)

    r0 = sid * ROWS_PER_TILE
    for t in range(ROWS_PER_TILE // ZCHUNK):
        pltpu.sync_copy(rows_v.at[pl.ds(0, ZCHUNK)],
                        acc.at[pl.ds(r0 + t * ZCHUNK, ZCHUNK)])

    plsc.subcore_barrier()

    pltpu.sync_copy(src_hbm.at[wid], src_v)
    pltpu.sync_copy(dst_hbm.at[wid], dst_v)

    def chunk_body(j, _):
        pltpu.async_copy(x_hbm.at[src_v.at[j]], rows_v, sem0).wait()
        pltpu.sync_copy(rows_v, acc.at[dst_v.at[j]], add=True)
        return 0

    lax.fori_loop(0, N_CHUNKS, chunk_body, 0)

    plsc.subcore_barrier()

    pltpu.sync_copy(acc.at[pl.ds(r0, ROWS_PER_TILE)],
                    out_hbm.at[cid, pl.ds(r0, ROWS_PER_TILE)])


BR = 1000


def _mlp_body(p_ref, x_ref, w1t_ref, b1_ref, w2t_ref, b2_ref, o_ref):
    h = p_ref[0] + p_ref[1] + x_ref[...]
    h1 = jnp.dot(h, w1t_ref[...], preferred_element_type=jnp.float32) + b1_ref[...]
    o_ref[...] = jnp.dot(h1, w2t_ref[...], preferred_element_type=jnp.float32) + b2_ref[...]


_mlp_call = pl.pallas_call(
    _mlp_body,
    out_shape=jax.ShapeDtypeStruct((N_NODES, D), jnp.float32),
    grid=(N_NODES // BR,),
    in_specs=[
        pl.BlockSpec((NC, BR, D), lambda i: (0, i, 0)),
        pl.BlockSpec((BR, D), lambda i: (i, 0)),
        pl.BlockSpec((D, D), lambda i: (0, 0)),
        pl.BlockSpec((1, D), lambda i: (0, 0)),
        pl.BlockSpec((D, D), lambda i: (0, 0)),
        pl.BlockSpec((1, D), lambda i: (0, 0)),
    ],
    out_specs=pl.BlockSpec((BR, D), lambda i: (i, 0)),
)


def kernel(x, edge_index, W1, b1, W2, b2):
    ei = edge_index.astype(jnp.int32)
    per_w = N_EDGES // NW
    dst = ei[0].reshape(NW, per_w)
    src = ei[1].reshape(NW, per_w)
    pad_rows = PAD_DST + jnp.arange(NW, dtype=jnp.int32)[:, None]
    dst = jnp.concatenate(
        [dst, jnp.broadcast_to(pad_rows, (NW, EPW - per_w))], axis=1)
    src = jnp.pad(src, ((0, 0), (0, EPW - per_w)), constant_values=0)
    dst = dst.reshape(NW, N_CHUNKS, CHUNK)
    src = src.reshape(NW, N_CHUNKS, CHUNK)
    partials = _sc_aggregate(x, src, dst)
    return _mlp_call(partials, x, W1.T, b1.reshape(1, D), W2.T, b2.reshape(1, D))


# speedup vs baseline: 2.9114x; 1.0063x over previous
import functools

import jax
import jax.numpy as jnp
from jax import lax
from jax.experimental import pallas as pl
from jax.experimental.pallas import tpu as pltpu
from jax.experimental.pallas import tpu_sc as plsc

N_NODES = 10000
N_EDGES = 320000
D = 128

NC = 2
NS = 16
NW = NC * NS
CHUNK = 100
N_CHUNKS = 100
EPW = N_CHUNKS * CHUNK
SUPER = 8
N_SUPER = N_CHUNKS // SUPER
N_PAD = 10240
ZCHUNK = 80
ROWS_PER_TILE = N_PAD // NS
PAD_DST = N_NODES

_mesh = plsc.VectorSubcoreMesh(core_axis_name="c", subcore_axis_name="s")


@functools.partial(
    pl.kernel,
    out_type=jax.ShapeDtypeStruct((NC, N_PAD, D), jnp.float32),
    mesh=_mesh,
    scratch_types=[
        pltpu.VMEM((N_CHUNKS, CHUNK), jnp.int32),
        pltpu.VMEM((N_CHUNKS, CHUNK), jnp.int32),
        pltpu.VMEM((CHUNK, D), jnp.float32),
        pltpu.VMEM_SHARED((N_PAD, D), jnp.float32),
        pltpu.SemaphoreType.DMA,
        pltpu.SemaphoreType.DMA,
    ],
)
def _sc_aggregate(x_hbm, src_hbm, dst_hbm, out_hbm, src_v, dst_v, rows_v, acc,
                  sem0, sem1):
    cid = lax.axis_index("c")
    sid = lax.axis_index("s")
    wid = cid * NS + sid

    zeros16 = jnp.zeros((16,), jnp.float32)

    def zero_body(i, _):
        rows_v[i // (D // 16), pl.ds((i % (D // 16)) * 16, 16)] = zeros16
        return 0

    lax.fori_loop(0, ZCHUNK * (D // 16), zero_body, 0)

    r0 = sid * ROWS_PER_TILE
    for t in range(ROWS_PER_TILE // ZCHUNK):
        pltpu.sync_copy(rows_v.at[pl.ds(0, ZCHUNK)],
                        acc.at[pl.ds(r0 + t * ZCHUNK, ZCHUNK)])

    plsc.subcore_barrier()

    pltpu.sync_copy(src_hbm.at[wid], src_v)
    pltpu.sync_copy(dst_hbm.at[wid], dst_v)

    def chunk_body(j, _):
        pltpu.async_copy(x_hbm.at[src_v.at[j]], rows_v, sem0).wait()
        pltpu.sync_copy(rows_v, acc.at[dst_v.at[j]], add=True)
        return 0

    lax.fori_loop(0, N_CHUNKS, chunk_body, 0)

    plsc.subcore_barrier()

    pltpu.sync_copy(acc.at[pl.ds(r0, ROWS_PER_TILE)],
                    out_hbm.at[cid, pl.ds(r0, ROWS_PER_TILE)])


BR = 1000


def _base_body(x_ref, w1t_ref, b1_ref, w2t_ref, b2_ref, base_ref, wf_ref):
    wf = jnp.dot(w1t_ref[...], w2t_ref[...], preferred_element_type=jnp.float32)
    wf_ref[...] = wf
    bias = jnp.dot(b1_ref[...], w2t_ref[...],
                   preferred_element_type=jnp.float32) + b2_ref[...]
    base_ref[...] = jnp.dot(x_ref[...], wf,
                            preferred_element_type=jnp.float32) + bias


_base_call = pl.pallas_call(
    _base_body,
    out_shape=(
        jax.ShapeDtypeStruct((N_NODES, D), jnp.float32),
        jax.ShapeDtypeStruct((D, D), jnp.float32),
    ),
    grid=(N_NODES // BR,),
    in_specs=[
        pl.BlockSpec((BR, D), lambda i: (i, 0)),
        pl.BlockSpec((D, D), lambda i: (0, 0)),
        pl.BlockSpec((1, D), lambda i: (0, 0)),
        pl.BlockSpec((D, D), lambda i: (0, 0)),
        pl.BlockSpec((1, D), lambda i: (0, 0)),
    ],
    out_specs=(
        pl.BlockSpec((BR, D), lambda i: (i, 0)),
        pl.BlockSpec((D, D), lambda i: (0, 0)),
    ),
)


def _final_body(p_ref, wf_ref, base_ref, o_ref):
    agg = p_ref[0] + p_ref[1]
    o_ref[...] = jnp.dot(agg, wf_ref[...],
                         preferred_element_type=jnp.float32) + base_ref[...]


_final_call = pl.pallas_call(
    _final_body,
    out_shape=jax.ShapeDtypeStruct((N_NODES, D), jnp.float32),
    grid=(N_NODES // BR,),
    in_specs=[
        pl.BlockSpec((NC, BR, D), lambda i: (0, i, 0)),
        pl.BlockSpec((D, D), lambda i: (0, 0)),
        pl.BlockSpec((BR, D), lambda i: (i, 0)),
    ],
    out_specs=pl.BlockSpec((BR, D), lambda i: (i, 0)),
)


def kernel(x, edge_index, W1, b1, W2, b2):
    ei = edge_index.astype(jnp.int32)
    per_w = N_EDGES // NW
    dst = ei[0].reshape(NW, per_w)
    src = ei[1].reshape(NW, per_w)
    pad_rows = PAD_DST + jnp.arange(NW, dtype=jnp.int32)[:, None]
    dst = jnp.concatenate(
        [dst, jnp.broadcast_to(pad_rows, (NW, EPW - per_w))], axis=1)
    src = jnp.pad(src, ((0, 0), (0, EPW - per_w)), constant_values=0)
    dst = dst.reshape(NW, N_CHUNKS, CHUNK)
    src = src.reshape(NW, N_CHUNKS, CHUNK)
    partials = _sc_aggregate(x, src, dst)
    base, wf = _base_call(x, W1.T, b1.reshape(1, D), W2.T, b2.reshape(1, D))
    return _final_call(partials, wf, base)
